# Initial kernel scaffold; baseline (speedup 1.0000x reference)
import jax
import jax.numpy as jnp
from jax.experimental import pallas as pl


def kernel(features, edge_index, W1, b1, W2, b2, W3, b3):
    raise NotImplementedError("write your pallas kernel here")


# speedup vs baseline: 3.9956x; 3.9956x over previous
import functools

import jax
import jax.numpy as jnp
from jax import lax
from jax.experimental import pallas as pl
from jax.experimental.pallas import tpu as pltpu
from jax.experimental.pallas import tpu_sc as plsc

N = 10000
E = 320000
NPAD = 10240
NROW = NPAD // 128
NB = NROW
CHP = 158
CHD = 158
EPAD = 16 * 2 * CHP * 64
ROWS_PER_TILE = NPAD // 16

_MESH = plsc.VectorSubcoreMesh(core_axis_name="c", subcore_axis_name="s")


def _deg_body(idx_hbm, out_hbm, idx_v, hist_v, red_v, ident_v, zb_v, obuf_v,
              acc_sh, sem):
    c = lax.axis_index("c")
    s = lax.axis_index("s")
    zeros16 = jnp.zeros((16,), jnp.float32)
    ones16 = jnp.full((16,), 1.0, jnp.float32)
    lanes = lax.iota(jnp.int32, 16)

    for p in range(3):
        for k in range(2):
            ident_v[p, pl.ds(k * 16, 16)] = lanes + (p * 32 + k * 16)
    for r in range(8):
        for k in range(8):
            zb_v[r, pl.ds(k * 16, 16)] = zeros16

    @pl.when(s < 12)
    def _zero():
        pltpu.sync_copy(zb_v, acc_sh.at[pl.ds(s * 8, 8)])

    plsc.subcore_barrier()

    pltpu.sync_copy(idx_hbm.at[c, s], idx_v)

    for p in range(3):
        lo = p * 4096

        def zh(j, carry):
            for r in range(16):
                hist_v[r, pl.ds(j * 16, 16)] = zeros16
            return carry

        lax.fori_loop(0, 256, zh, 0)

        def hb(j, carry):
            for k in range(8):
                idx16 = idx_v[j, pl.ds(k * 16, 16)]
                m = (idx16 >= lo) & (idx16 < lo + 4096)
                col = jnp.where(m, idx16 - lo, 0)
                plsc.addupdate_scatter(hist_v, [lanes, col], ones16, mask=m)
            return carry

        lax.fori_loop(0, CHD, hb, 0)

        def rb(r, carry):
            for k in range(8):
                a = hist_v[0, pl.ds(r * 128 + k * 16, 16)]
                for row in range(1, 16):
                    a = a + hist_v[row, pl.ds(r * 128 + k * 16, 16)]
                red_v[r, pl.ds(k * 16, 16)] = a
            return carry

        lax.fori_loop(0, 32, rb, 0)
        pltpu.sync_copy(red_v, acc_sh.at[ident_v.at[p]], add=True)

    plsc.subcore_barrier()

    @pl.when(s < 10)
    def _copy_out():
        pltpu.sync_copy(acc_sh.at[pl.ds(s * 8, 8)], obuf_v)
        pltpu.sync_copy(obuf_v, out_hbm.at[c, pl.ds(s * 8, 8)])


_sc_degrees = functools.partial(
    pl.kernel,
    out_type=jax.ShapeDtypeStruct((2, NROW, 128), jnp.float32),
    mesh=_MESH,
    scratch_types=[
        pltpu.VMEM((CHD, 128), jnp.int32),
        pltpu.VMEM((16, 4096), jnp.float32),
        pltpu.VMEM((32, 128), jnp.float32),
        pltpu.VMEM((3, 32), jnp.int32),
        pltpu.VMEM((8, 128), jnp.float32),
        pltpu.VMEM((8, 128), jnp.float32),
        pltpu.VMEM_SHARED((96, 128), jnp.float32),
        pltpu.SemaphoreType.DMA,
    ],
    compiler_params=pltpu.CompilerParams(needs_layout_passes=False),
)(_deg_body)


def _prop_body(src_hbm, dst_hbm, m_hbm, out_hbm, src_v, dst_v, buf0, buf1,
               zb_v, acc_sh, gs0, gs1, ss0, ss1):
    c = lax.axis_index("c")
    s = lax.axis_index("s")

    zeros16 = jnp.zeros((16,), jnp.float32)
    for i in range(16):
        for k in range(4):
            zb_v[i, pl.ds(k * 16, 16)] = zeros16

    def zbody(k, carry):
        pltpu.sync_copy(zb_v, acc_sh.at[pl.ds(s * ROWS_PER_TILE + k * 16, 16)])
        return carry

    lax.fori_loop(0, ROWS_PER_TILE // 16, zbody, 0)
    plsc.subcore_barrier()

    def gather(j, buf, sem):
        return pltpu.async_copy(m_hbm.at[src_v.at[j]], buf, sem)

    def scat(j, buf, sem):
        return pltpu.async_copy(buf, acc_sh.at[dst_v.at[j]], sem, add=True)

    for ph in range(2):
        pltpu.sync_copy(src_hbm.at[c, s, ph], src_v)
        pltpu.sync_copy(dst_hbm.at[s, ph], dst_v)

        def pair(j0, j1):
            g0 = gather(j0, buf0, gs0)
            g1 = gather(j1, buf1, gs1)
            g0.wait()
            s0 = scat(j0, buf0, ss0)
            g1.wait()
            s1 = scat(j1, buf1, ss1)
            s0.wait()
            s1.wait()

        def body(i, carry):
            pair(2 * i, 2 * i + 1)
            return carry

        lax.fori_loop(0, CHP // 2, body, 0)

    plsc.subcore_barrier()

    def obody(k, carry):
        pltpu.sync_copy(acc_sh.at[pl.ds(s * ROWS_PER_TILE + k * 64, 64)], buf0)
        pltpu.sync_copy(buf0, out_hbm.at[pl.ds(c * NPAD + s * ROWS_PER_TILE + k * 64, 64)])
        return carry

    lax.fori_loop(0, ROWS_PER_TILE // 64, obody, 0)


_sc_prop = functools.partial(
    pl.kernel,
    out_type=jax.ShapeDtypeStruct((2 * NPAD, 64), jnp.float32),
    mesh=_MESH,
    scratch_types=[
        pltpu.VMEM((CHP, 64), jnp.int32),
        pltpu.VMEM((CHP, 64), jnp.int32),
        pltpu.VMEM((64, 64), jnp.float32),
        pltpu.VMEM((64, 64), jnp.float32),
        pltpu.VMEM((16, 64), jnp.float32),
        pltpu.VMEM_SHARED((NPAD, 64), jnp.float32),
        pltpu.SemaphoreType.DMA,
        pltpu.SemaphoreType.DMA,
        pltpu.SemaphoreType.DMA,
        pltpu.SemaphoreType.DMA,
    ],
    compiler_params=pltpu.CompilerParams(use_tc_tiling_on_sc=False),
)(_prop_body)


def _norm(deg):
    return lax.rsqrt(jnp.maximum(deg, 1.0))


def _tc_pre_body(x_ref, w_ref, dego_ref, m_ref):
    ns = _norm(dego_ref[...])
    m_ref[...] = ns * jnp.dot(x_ref[...], w_ref[0], preferred_element_type=jnp.float32)


def _tc_pre(x, w_split, deg_o):
    return pl.pallas_call(
        _tc_pre_body,
        grid=(2, NB),
        in_specs=[
            pl.BlockSpec((128, 128), lambda c, r: (r, 0)),
            pl.BlockSpec((1, 128, 64), lambda c, r: (c, 0, 0)),
            pl.BlockSpec((128, 1), lambda c, r: (r, 0)),
        ],
        out_specs=pl.BlockSpec((128, 64), lambda c, r: (c * NB + r, 0)),
        out_shape=jax.ShapeDtypeStruct((2 * NPAD, 64), jnp.float32),
    )(x, w_split, deg_o)


def _tc_mid_body(p0_ref, p1_ref, w_ref, b_ref, degi_ref, dego_ref, m_ref):
    nd = _norm(degi_ref[...])
    ns = _norm(dego_ref[...])
    b = b_ref[...]
    h0 = jax.nn.relu(nd * p0_ref[...] + b[:, 0:64])
    h1 = jax.nn.relu(nd * p1_ref[...] + b[:, 64:128])
    w = w_ref[0]
    acc = jnp.dot(h0, w[0:64, :], preferred_element_type=jnp.float32)
    acc = acc + jnp.dot(h1, w[64:128, :], preferred_element_type=jnp.float32)
    m_ref[...] = ns * acc


def _tc_mid(p, w_split, b, deg_i, deg_o):
    return pl.pallas_call(
        _tc_mid_body,
        grid=(2, NB),
        in_specs=[
            pl.BlockSpec((128, 64), lambda c, r: (r, 0)),
            pl.BlockSpec((128, 64), lambda c, r: (NB + r, 0)),
            pl.BlockSpec((1, 128, 64), lambda c, r: (c, 0, 0)),
            pl.BlockSpec((1, 128), lambda c, r: (0, 0)),
            pl.BlockSpec((128, 1), lambda c, r: (r, 0)),
            pl.BlockSpec((128, 1), lambda c, r: (r, 0)),
        ],
        out_specs=pl.BlockSpec((128, 64), lambda c, r: (c * NB + r, 0)),
        out_shape=jax.ShapeDtypeStruct((2 * NPAD, 64), jnp.float32),
    )(p, p, w_split, b.reshape(1, 128), deg_i, deg_o)


def _tc_mid3_body(p_ref, b_ref, degi_ref, dego_ref, m_ref):
    nd = _norm(degi_ref[...])
    ns = _norm(dego_ref[...])
    m_ref[...] = ns * jax.nn.relu(nd * p_ref[...] + b_ref[0])


def _tc_mid3(p, b, deg_i, deg_o):
    return pl.pallas_call(
        _tc_mid3_body,
        grid=(2, NB),
        in_specs=[
            pl.BlockSpec((128, 64), lambda c, r: (c * NB + r, 0)),
            pl.BlockSpec((1, 1, 64), lambda c, r: (c, 0, 0)),
            pl.BlockSpec((128, 1), lambda c, r: (r, 0)),
            pl.BlockSpec((128, 1), lambda c, r: (r, 0)),
        ],
        out_specs=pl.BlockSpec((128, 64), lambda c, r: (c * NB + r, 0)),
        out_shape=jax.ShapeDtypeStruct((2 * NPAD, 64), jnp.float32),
    )(p, b.reshape(2, 1, 64), deg_i, deg_o)


def _tc_fin_body(p0_ref, p1_ref, w_ref, b_ref, degi_ref, o_ref):
    nd = _norm(degi_ref[...])
    agg = jnp.concatenate([p0_ref[...], p1_ref[...]], axis=1)
    o_ref[...] = jnp.dot(nd * agg, w_ref[...], preferred_element_type=jnp.float32) + b_ref[...]


def _tc_fin(p, w, b, deg_i):
    return pl.pallas_call(
        _tc_fin_body,
        grid=(NB,),
        in_specs=[
            pl.BlockSpec((128, 64), lambda r: (r, 0)),
            pl.BlockSpec((128, 64), lambda r: (NB + r, 0)),
            pl.BlockSpec((128, 64), lambda r: (0, 0)),
            pl.BlockSpec((1, 64), lambda r: (0, 0)),
            pl.BlockSpec((128, 1), lambda r: (r, 0)),
        ],
        out_specs=pl.BlockSpec((128, 64), lambda r: (r, 0)),
        out_shape=jax.ShapeDtypeStruct((NPAD, 64), jnp.float32),
    )(p, p, w, b.reshape(1, 64), deg_i)


@jax.jit
def kernel(features, edge_index, W1, b1, W2, b2, W3, b3):
    src = edge_index[0].astype(jnp.int32)
    dst = edge_index[1].astype(jnp.int32)
    pad = jnp.full((EPAD - E,), N, jnp.int32)
    srcp = jnp.concatenate([src, pad])
    dstp = jnp.concatenate([dst, pad])
    prop_src = jnp.stack([srcp, srcp + NPAD]).reshape(2, 16, 2, CHP, 64)
    prop_dst = dstp.reshape(16, 2, CHP, 64)
    deg_idx = jnp.stack([srcp, dstp]).reshape(2, 16, CHD, 128)

    deg = _sc_degrees(deg_idx)
    deg_o = deg[0].reshape(NPAD, 1)
    deg_i = deg[1].reshape(NPAD, 1)

    x = jnp.concatenate([features, jnp.zeros((NPAD - N, features.shape[1]), jnp.float32)])
    w1s = jnp.stack([W1[:, :64], W1[:, 64:]])
    w2s = jnp.stack([W2[:, :64], W2[:, 64:]])

    m1 = _tc_pre(x, w1s, deg_o)
    p1 = _sc_prop(prop_src, prop_dst, m1)
    m2 = _tc_mid(p1, w2s, b1, deg_i, deg_o)
    p2 = _sc_prop(prop_src, prop_dst, m2)
    m3 = _tc_mid3(p2, b2, deg_i, deg_o)
    p3 = _sc_prop(prop_src, prop_dst, m3)
    out = _tc_fin(p3, W3, b3, deg_i)
    return out[:N]
